# baseline (speedup 1.0000x reference)
import functools

import jax
import jax.numpy as jnp
from jax.experimental import pallas as pl

_NUM_CLASSES = 22
_INLIER_THRESHOLD = 0.9
_LABEL_THRESHOLD = 500.0
_THRESHOLD_VOTE = 5.0
_THRESHOLD_PERCENTAGE = 0.05
_SKIP = 16


def _rn_bf16(x):
    b = jax.lax.bitcast_convert_type(x, jnp.uint32)
    b = (b + jnp.uint32(0x7FFF) + ((b >> 16) & jnp.uint32(1))) & jnp.uint32(0xFFFF0000)
    return jax.lax.bitcast_convert_type(b, jnp.float32)


def _gather_kernel(vrow_ref, lrow_ref, vout_ref, lout_ref, *, nc3):
    vout_ref[...] = vrow_ref[:, 0:nc3]
    lout_ref[...] = lrow_ref[:, 0:1]


def _hough_kernel(labP_ref, vall_ref, labF_ref, extT_ref,
                  poses_ref, meta_ref, boxT_ref, poseT_ref, tgtT_ref, wtT_ref,
                  *, P, C, nw, s):
    f32 = jnp.float32
    lab = labP_ref[...]
    cls_lane = jax.lax.broadcasted_iota(jnp.int32, (1, C), 1)
    onehot = (lab == cls_lane).astype(f32)

    vall = vall_ref[...]
    ch_lane = jax.lax.broadcasted_iota(jnp.int32, (1, 3 * C), 1)
    lab3 = lab * 3
    dnx_r = jnp.sum(jnp.where(ch_lane == lab3, vall, 0.0), axis=1, keepdims=True)
    dny_r = jnp.sum(jnp.where(ch_lane == lab3 + 1, vall, 0.0), axis=1, keepdims=True)
    dnorm = jnp.sqrt(dnx_r * dnx_r + dny_r * dny_r)
    dnx = _rn_bf16(dnx_r / (dnorm + 1e-8))
    dny = _rn_bf16(dny_r / (dnorm + 1e-8))

    p_iota = jax.lax.broadcasted_iota(jnp.int32, (P, 1), 0)
    pixx = ((p_iota % nw) * s).astype(f32)
    pixy = ((p_iota // nw) * s).astype(f32)
    g_iota = jax.lax.broadcasted_iota(jnp.int32, (1, P), 1)
    candx = ((g_iota % nw) * s).astype(f32) + (s / 2.0)
    candy = ((g_iota // nw) * s).astype(f32) + (s / 2.0)

    dx = candx - pixx
    dy = candy - pixy
    vnorm = jnp.sqrt(dx * dx + dy * dy)
    vecnx = _rn_bf16(dx / (vnorm + 1e-8))
    vecny = _rn_bf16(dy / (vnorm + 1e-8))
    dot = vecnx * dnx + vecny * dny
    inlier = (dot > _INLIER_THRESHOLD).astype(f32)

    votes = jax.lax.dot_general(inlier, onehot, (((0,), (0,)), ((), ())),
                                preferred_element_type=f32)

    count = jnp.max(votes, axis=0, keepdims=True)
    grow = jax.lax.broadcasted_iota(jnp.int32, (P, C), 0)
    best = jnp.min(jnp.where(votes == count, grow, P),
                   axis=0, keepdims=True)
    cxf = ((best % nw) * s + s // 2).astype(f32)
    cyf = ((best // nw) * s + s // 2).astype(f32)

    npix_sub = jnp.sum(onehot, axis=0, keepdims=True)
    dz = jnp.sum(jnp.where(ch_lane == lab3 + 2, vall, 0.0), axis=1, keepdims=True)
    depth_sum = jnp.sum(onehot * dz, axis=0, keepdims=True)

    labF = labF_ref[...]
    npix_full = jnp.zeros((1, C), f32)
    for c in range(C):
        s_c = jnp.sum((labF == c).astype(f32))
        npix_full = npix_full + jnp.where(cls_lane == c, s_c, 0.0)

    frac = count / jnp.maximum(npix_sub, 1.0)
    validf = (jnp.logical_and(
        jnp.logical_and(npix_full > _LABEL_THRESHOLD, count > _THRESHOLD_VOTE),
        frac > _THRESHOLD_PERCENTAGE)).astype(f32)
    depth = jnp.abs(depth_sum / jnp.maximum(npix_sub, 1.0)) + 0.5

    e0 = extT_ref[0:1, :]
    e1 = extT_ref[1:2, :]
    e2 = extT_ref[2:3, :]
    ext = jnp.sqrt(e0 * e0 + e1 * e1 + e2 * e2)
    fx = meta_ref[0, 0]
    px0 = meta_ref[0, 2]
    fy = meta_ref[0, 4]
    py0 = meta_ref[0, 5]
    bbw = ext * fx / depth
    bbh = ext * fy / depth
    tx = (cxf - px0) * depth / (jnp.abs(fx) + 1e-3)
    ty = (cyf - py0) * depth / (jnp.abs(fy) + 1e-3)
    q0 = poses_ref[0, 6]
    q1 = poses_ref[0, 7]
    q2 = poses_ref[0, 8]
    q3 = poses_ref[0, 9]

    clsf = cls_lane.astype(f32)
    rows8 = jax.lax.broadcasted_iota(jnp.int32, (8, 1), 0)

    def sel(k, val):
        return jnp.where(rows8 == k, val, 0.0)

    boxT_ref[...] = (sel(1, clsf)
                     + sel(2, (cxf - bbw / 2.0) * validf)
                     + sel(3, (cyf - bbh / 2.0) * validf)
                     + sel(4, (cxf + bbw / 2.0) * validf)
                     + sel(5, (cyf + bbh / 2.0) * validf)
                     + sel(6, count))
    poseT_ref[...] = (sel(0, q0 * validf) + sel(1, q1 * validf)
                      + sel(2, q2 * validf) + sel(3, q3 * validf)
                      + sel(4, tx * validf) + sel(5, ty * validf)
                      + sel(6, depth * validf))

    sub4c = jax.lax.broadcasted_iota(jnp.int32, (4 * C, 1), 0)
    qmod = sub4c % 4
    qcol = jnp.where(qmod == 0, q0,
                     jnp.where(qmod == 1, q1,
                               jnp.where(qmod == 2, q2, q3)))
    clsmatch = ((sub4c // 4) == cls_lane).astype(f32)
    tgtT_ref[...] = qcol * validf * clsmatch
    wtT_ref[...] = clsmatch * validf


def kernel(label_2d, vertex_pred, extents, poses, meta_data):
    B, H, W = label_2d.shape
    C = extents.shape[0]
    s = _SKIP
    nw = W // s
    nh = H // s
    P = nh * nw
    nc3 = 3 * C

    f32 = jnp.float32
    vview = vertex_pred.reshape(H * nw, s * nc3)
    lview = label_2d.reshape(H * nw, s)
    vall, labP = pl.pallas_call(
        functools.partial(_gather_kernel, nc3=nc3),
        grid=(nh,),
        in_specs=[
            pl.BlockSpec((nw, s * nc3), lambda i: (i * s, 0)),
            pl.BlockSpec((nw, s), lambda i: (i * s, 0)),
        ],
        out_specs=[
            pl.BlockSpec((nw, nc3), lambda i: (i, 0)),
            pl.BlockSpec((nw, 1), lambda i: (i, 0)),
        ],
        out_shape=[
            jax.ShapeDtypeStruct((P, nc3), f32),
            jax.ShapeDtypeStruct((P, 1), jnp.int32),
        ],
    )(vview, lview)

    labF = label_2d.reshape(-1, 128)
    extT = jnp.pad(extents.T, ((0, 5), (0, 0)))

    out_shapes = [
        jax.ShapeDtypeStruct((8, C), f32),
        jax.ShapeDtypeStruct((8, C), f32),
        jax.ShapeDtypeStruct((4 * C, C), f32),
        jax.ShapeDtypeStruct((4 * C, C), f32),
    ]
    boxT, poseT, tgtT, wtT = pl.pallas_call(
        functools.partial(_hough_kernel, P=P, C=C, nw=nw, s=s),
        out_shape=out_shapes,
    )(labP, vall, labF, extT, poses, meta_data)

    top_box = boxT.T[1:C, 0:7]
    top_pose = poseT.T[1:C, 0:7]
    top_target = tgtT.T[1:C, :]
    top_weight = wtT.T[1:C, :]
    top_domain = jnp.zeros((C - 1,), f32)
    return (top_box, top_pose, top_target, top_weight, top_domain)


# speedup vs baseline: 5.0024x; 5.0024x over previous
import functools

import jax
import jax.numpy as jnp
from jax.experimental import pallas as pl

_NUM_CLASSES = 22
_INLIER_THRESHOLD = 0.9
_LABEL_THRESHOLD = 500.0
_THRESHOLD_VOTE = 5.0
_THRESHOLD_PERCENTAGE = 0.05
_SKIP = 16


def _rn_bf16(x):
    b = jax.lax.bitcast_convert_type(x, jnp.uint32)
    b = (b + jnp.uint32(0x7FFF) + ((b >> 16) & jnp.uint32(1))) & jnp.uint32(0xFFFF0000)
    return jax.lax.bitcast_convert_type(b, jnp.float32)


def _gather_kernel(vrow_ref, lrow_ref, vout_ref, lout_ref, *, nc3):
    vout_ref[...] = vrow_ref[:, 0:nc3]
    lout_ref[...] = lrow_ref[:, 0:1]


def _hough_kernel(labP_ref, vall_ref, labF_ref, extT_ref,
                  poses_ref, meta_ref, boxT_ref, poseT_ref, tgtT_ref, wtT_ref,
                  *, P, C, nw, s):
    f32 = jnp.float32
    lab = labP_ref[...]
    cls_lane = jax.lax.broadcasted_iota(jnp.int32, (1, C), 1)
    onehot = (lab == cls_lane).astype(f32)

    vall = vall_ref[...]
    ch_lane = jax.lax.broadcasted_iota(jnp.int32, (1, 3 * C), 1)
    lab3 = lab * 3
    dnx_r = jnp.sum(jnp.where(ch_lane == lab3, vall, 0.0), axis=1, keepdims=True)
    dny_r = jnp.sum(jnp.where(ch_lane == lab3 + 1, vall, 0.0), axis=1, keepdims=True)
    dnorm = jnp.sqrt(dnx_r * dnx_r + dny_r * dny_r)
    dnx = _rn_bf16(dnx_r / (dnorm + 1e-8))
    dny = _rn_bf16(dny_r / (dnorm + 1e-8))

    p_iota = jax.lax.broadcasted_iota(jnp.int32, (P, 1), 0)
    pixx = ((p_iota % nw) * s).astype(f32)
    pixy = ((p_iota // nw) * s).astype(f32)
    g_iota = jax.lax.broadcasted_iota(jnp.int32, (1, P), 1)
    candx = ((g_iota % nw) * s).astype(f32) + (s / 2.0)
    candy = ((g_iota // nw) * s).astype(f32) + (s / 2.0)

    dx = candx - pixx
    dy = candy - pixy
    vnorm = jnp.sqrt(dx * dx + dy * dy)
    vecnx = _rn_bf16(dx / (vnorm + 1e-8))
    vecny = _rn_bf16(dy / (vnorm + 1e-8))
    dot = vecnx * dnx + vecny * dny
    inlier = (dot > _INLIER_THRESHOLD).astype(f32)

    votes = jax.lax.dot_general(inlier, onehot, (((0,), (0,)), ((), ())),
                                preferred_element_type=f32)

    count = jnp.max(votes, axis=0, keepdims=True)
    grow = jax.lax.broadcasted_iota(jnp.int32, (P, C), 0)
    best = jnp.min(jnp.where(votes == count, grow, P),
                   axis=0, keepdims=True)
    cxf = ((best % nw) * s + s // 2).astype(f32)
    cyf = ((best // nw) * s + s // 2).astype(f32)

    npix_sub = jnp.sum(onehot, axis=0, keepdims=True)
    dz = jnp.sum(jnp.where(ch_lane == lab3 + 2, vall, 0.0), axis=1, keepdims=True)
    depth_sum = jnp.sum(onehot * dz, axis=0, keepdims=True)

    labF = labF_ref[...]
    npix_full = jnp.zeros((1, C), f32)
    for c in range(C):
        s_c = jnp.sum((labF == c).astype(f32))
        npix_full = npix_full + jnp.where(cls_lane == c, s_c, 0.0)

    frac = count / jnp.maximum(npix_sub, 1.0)
    validf = (jnp.logical_and(
        jnp.logical_and(npix_full > _LABEL_THRESHOLD, count > _THRESHOLD_VOTE),
        frac > _THRESHOLD_PERCENTAGE)).astype(f32)
    depth = jnp.abs(depth_sum / jnp.maximum(npix_sub, 1.0)) + 0.5

    e0 = extT_ref[0:1, :]
    e1 = extT_ref[1:2, :]
    e2 = extT_ref[2:3, :]
    ext = jnp.sqrt(e0 * e0 + e1 * e1 + e2 * e2)
    fx = meta_ref[0, 0]
    px0 = meta_ref[0, 2]
    fy = meta_ref[0, 4]
    py0 = meta_ref[0, 5]
    bbw = ext * fx / depth
    bbh = ext * fy / depth
    tx = (cxf - px0) * depth / (jnp.abs(fx) + 1e-3)
    ty = (cyf - py0) * depth / (jnp.abs(fy) + 1e-3)
    q0 = poses_ref[0, 6]
    q1 = poses_ref[0, 7]
    q2 = poses_ref[0, 8]
    q3 = poses_ref[0, 9]

    clsf = cls_lane.astype(f32)
    rows8 = jax.lax.broadcasted_iota(jnp.int32, (8, 1), 0)

    def sel(k, val):
        return jnp.where(rows8 == k, val, 0.0)

    boxT_ref[...] = (sel(1, clsf)
                     + sel(2, (cxf - bbw / 2.0) * validf)
                     + sel(3, (cyf - bbh / 2.0) * validf)
                     + sel(4, (cxf + bbw / 2.0) * validf)
                     + sel(5, (cyf + bbh / 2.0) * validf)
                     + sel(6, count))
    poseT_ref[...] = (sel(0, q0 * validf) + sel(1, q1 * validf)
                      + sel(2, q2 * validf) + sel(3, q3 * validf)
                      + sel(4, tx * validf) + sel(5, ty * validf)
                      + sel(6, depth * validf))

    sub4c = jax.lax.broadcasted_iota(jnp.int32, (4 * C, 1), 0)
    qmod = sub4c % 4
    qcol = jnp.where(qmod == 0, q0,
                     jnp.where(qmod == 1, q1,
                               jnp.where(qmod == 2, q2, q3)))
    clsmatch = ((sub4c // 4) == cls_lane).astype(f32)
    tgtT_ref[...] = qcol * validf * clsmatch
    wtT_ref[...] = clsmatch * validf


def kernel(label_2d, vertex_pred, extents, poses, meta_data):
    B, H, W = label_2d.shape
    C = extents.shape[0]
    s = _SKIP
    nw = W // s
    nh = H // s
    P = nh * nw
    nc3 = 3 * C

    f32 = jnp.float32
    vall = vertex_pred[0, ::s, ::s, :].reshape(P, nc3)
    labP = label_2d[0, ::s, ::s].reshape(P, 1)

    labF = label_2d.reshape(-1, 128)
    extT = jnp.pad(extents.T, ((0, 5), (0, 0)))

    out_shapes = [
        jax.ShapeDtypeStruct((8, C), f32),
        jax.ShapeDtypeStruct((8, C), f32),
        jax.ShapeDtypeStruct((4 * C, C), f32),
        jax.ShapeDtypeStruct((4 * C, C), f32),
    ]
    boxT, poseT, tgtT, wtT = pl.pallas_call(
        functools.partial(_hough_kernel, P=P, C=C, nw=nw, s=s),
        out_shape=out_shapes,
    )(labP, vall, labF, extT, poses, meta_data)

    top_box = boxT.T[1:C, 0:7]
    top_pose = poseT.T[1:C, 0:7]
    top_target = tgtT.T[1:C, :]
    top_weight = wtT.T[1:C, :]
    top_domain = jnp.zeros((C - 1,), f32)
    return (top_box, top_pose, top_target, top_weight, top_domain)


# speedup vs baseline: 5.0055x; 1.0006x over previous
import functools

import jax
import jax.numpy as jnp
from jax.experimental import pallas as pl

_NUM_CLASSES = 22
_INLIER_THRESHOLD = 0.9
_LABEL_THRESHOLD = 500.0
_THRESHOLD_VOTE = 5.0
_THRESHOLD_PERCENTAGE = 0.05
_SKIP = 16


def _rn_bf16(x):
    b = jax.lax.bitcast_convert_type(x, jnp.uint32)
    b = (b + jnp.uint32(0x7FFF) + ((b >> 16) & jnp.uint32(1))) & jnp.uint32(0xFFFF0000)
    return jax.lax.bitcast_convert_type(b, jnp.float32)


def _hough_kernel(labP_ref, vall_ref, labF_ref, extT_ref,
                  poses_ref, meta_ref, boxT_ref, poseT_ref, tgtT_ref, wtT_ref,
                  *, P, C, nw, s):
    f32 = jnp.float32
    lab = labP_ref[...]
    cls_lane = jax.lax.broadcasted_iota(jnp.int32, (1, C), 1)
    onehot = (lab == cls_lane).astype(f32)

    vall = vall_ref[...]
    ch_lane = jax.lax.broadcasted_iota(jnp.int32, (1, 3 * C), 1)
    lab3 = lab * 3
    dnx_r = jnp.sum(jnp.where(ch_lane == lab3, vall, 0.0), axis=1, keepdims=True)
    dny_r = jnp.sum(jnp.where(ch_lane == lab3 + 1, vall, 0.0), axis=1, keepdims=True)
    dnorm = jnp.sqrt(dnx_r * dnx_r + dny_r * dny_r)
    dnx = _rn_bf16(dnx_r / (dnorm + 1e-8))
    dny = _rn_bf16(dny_r / (dnorm + 1e-8))

    p_iota = jax.lax.broadcasted_iota(jnp.int32, (P, 1), 0)
    pixx = ((p_iota % nw) * s).astype(f32)
    pixy = ((p_iota // nw) * s).astype(f32)
    g_iota = jax.lax.broadcasted_iota(jnp.int32, (1, P), 1)
    candx = ((g_iota % nw) * s).astype(f32) + (s / 2.0)
    candy = ((g_iota // nw) * s).astype(f32) + (s / 2.0)

    dx = candx - pixx
    dy = candy - pixy
    vnorm = jnp.sqrt(dx * dx + dy * dy)
    vecnx = _rn_bf16(dx / (vnorm + 1e-8))
    vecny = _rn_bf16(dy / (vnorm + 1e-8))
    dot = vecnx * dnx + vecny * dny
    inlier = (dot > _INLIER_THRESHOLD).astype(f32)

    votes = jax.lax.dot_general(inlier, onehot, (((0,), (0,)), ((), ())),
                                preferred_element_type=f32)

    count = jnp.max(votes, axis=0, keepdims=True)
    grow = jax.lax.broadcasted_iota(jnp.int32, (P, C), 0)
    best = jnp.min(jnp.where(votes == count, grow, P),
                   axis=0, keepdims=True)
    cxf = ((best % nw) * s + s // 2).astype(f32)
    cyf = ((best // nw) * s + s // 2).astype(f32)

    npix_sub = jnp.sum(onehot, axis=0, keepdims=True)
    dz = jnp.sum(jnp.where(ch_lane == lab3 + 2, vall, 0.0), axis=1, keepdims=True)
    depth_sum = jnp.sum(onehot * dz, axis=0, keepdims=True)

    labF = labF_ref[...]
    npix_full = jnp.zeros((1, C), f32)
    for c in range(C):
        s_c = jnp.sum((labF == c).astype(f32))
        npix_full = npix_full + jnp.where(cls_lane == c, s_c, 0.0)

    frac = count / jnp.maximum(npix_sub, 1.0)
    validf = (jnp.logical_and(
        jnp.logical_and(npix_full > _LABEL_THRESHOLD, count > _THRESHOLD_VOTE),
        frac > _THRESHOLD_PERCENTAGE)).astype(f32)
    depth = jnp.abs(depth_sum / jnp.maximum(npix_sub, 1.0)) + 0.5

    e0 = extT_ref[0:1, :]
    e1 = extT_ref[1:2, :]
    e2 = extT_ref[2:3, :]
    ext = jnp.sqrt(e0 * e0 + e1 * e1 + e2 * e2)
    fx = meta_ref[0, 0]
    px0 = meta_ref[0, 2]
    fy = meta_ref[0, 4]
    py0 = meta_ref[0, 5]
    bbw = ext * fx / depth
    bbh = ext * fy / depth
    tx = (cxf - px0) * depth / (jnp.abs(fx) + 1e-3)
    ty = (cyf - py0) * depth / (jnp.abs(fy) + 1e-3)
    q0 = poses_ref[0, 6]
    q1 = poses_ref[0, 7]
    q2 = poses_ref[0, 8]
    q3 = poses_ref[0, 9]

    clsf = cls_lane.astype(f32)
    rows8 = jax.lax.broadcasted_iota(jnp.int32, (8, 1), 0)

    def sel(k, val):
        return jnp.where(rows8 == k, val, 0.0)

    boxT_ref[...] = (sel(1, clsf)
                     + sel(2, (cxf - bbw / 2.0) * validf)
                     + sel(3, (cyf - bbh / 2.0) * validf)
                     + sel(4, (cxf + bbw / 2.0) * validf)
                     + sel(5, (cyf + bbh / 2.0) * validf)
                     + sel(6, count))
    poseT_ref[...] = (sel(0, q0 * validf) + sel(1, q1 * validf)
                      + sel(2, q2 * validf) + sel(3, q3 * validf)
                      + sel(4, tx * validf) + sel(5, ty * validf)
                      + sel(6, depth * validf))

    sub4c = jax.lax.broadcasted_iota(jnp.int32, (4 * C, 1), 0)
    qmod = sub4c % 4
    qcol = jnp.where(qmod == 0, q0,
                     jnp.where(qmod == 1, q1,
                               jnp.where(qmod == 2, q2, q3)))
    clsmatch = ((sub4c // 4) == cls_lane).astype(f32)
    tgtT_ref[...] = qcol * validf * clsmatch
    wtT_ref[...] = clsmatch * validf


def kernel(label_2d, vertex_pred, extents, poses, meta_data):
    B, H, W = label_2d.shape
    C = extents.shape[0]
    s = _SKIP
    nw = W // s
    nh = H // s
    P = nh * nw
    nc3 = 3 * C

    f32 = jnp.float32
    vall = vertex_pred[0, ::s, ::s, :].reshape(P, nc3)
    labP = label_2d[0, ::s, ::s].reshape(P, 1)

    labF = label_2d.reshape(-1, 128)
    extT = jnp.pad(extents.T, ((0, 5), (0, 0)))

    out_shapes = [
        jax.ShapeDtypeStruct((8, C), f32),
        jax.ShapeDtypeStruct((8, C), f32),
        jax.ShapeDtypeStruct((4 * C, C), f32),
        jax.ShapeDtypeStruct((4 * C, C), f32),
    ]
    boxT, poseT, tgtT, wtT = pl.pallas_call(
        functools.partial(_hough_kernel, P=P, C=C, nw=nw, s=s),
        out_shape=out_shapes,
    )(labP, vall, labF, extT, poses, meta_data)

    top_box = boxT.T[1:C, 0:7]
    top_pose = poseT.T[1:C, 0:7]
    top_target = tgtT.T[1:C, :]
    top_weight = wtT.T[1:C, :]
    top_domain = jnp.zeros((C - 1,), f32)
    return (top_box, top_pose, top_target, top_weight, top_domain)
